# Initial kernel scaffold; baseline (speedup 1.0000x reference)
import jax
import jax.numpy as jnp
from jax.experimental import pallas as pl


def kernel(id_batch, neigh_cell, neigh_drug, neigh_gene, drug_features, gene_features, cell_table, W_drug, b_drug, W_gene, b_gene, att):
    raise NotImplementedError("write your pallas kernel here")


# speedup vs baseline: 4.5844x; 4.5844x over previous
import functools

import jax
import jax.numpy as jnp
from jax import lax
from jax.experimental import pallas as pl
from jax.experimental.pallas import tpu as pltpu
from jax.experimental.pallas import tpu_sc as plsc


def _proj_body(x_ref, w_ref, b_ref, o_ref):
    o_ref[...] = (
        jnp.dot(x_ref[...], w_ref[...], preferred_element_type=jnp.float32)
        + b_ref[...]
    )


def _project(x, w, b):
    n, fdim = x.shape
    d = w.shape[1]
    blk = 2000
    assert n % blk == 0
    return pl.pallas_call(
        _proj_body,
        grid=(n // blk,),
        in_specs=[
            pl.BlockSpec((blk, fdim), lambda i: (i, 0)),
            pl.BlockSpec((fdim, d), lambda i: (0, 0)),
            pl.BlockSpec((1, d), lambda i: (0, 0)),
        ],
        out_specs=pl.BlockSpec((blk, d), lambda i: (i, 0)),
        out_shape=jax.ShapeDtypeStruct((n, d), jnp.float32),
    )(x, w, b.reshape(1, d))


def _make_sc_gather(Bn, d, s_per):
    info = plsc.get_sparse_core_info()
    NW = info.num_cores * info.num_subcores
    CB = 8
    G = CB * s_per
    rows_w = Bn // NW
    nchunks = rows_w // CB
    assert Bn % (NW * CB) == 0
    mesh = plsc.VectorSubcoreMesh(core_axis_name="c", subcore_axis_name="s")

    @functools.partial(
        pl.kernel,
        mesh=mesh,
        out_type=[jax.ShapeDtypeStruct((Bn, d), jnp.float32)] * 4,
        scratch_types=[
            pltpu.VMEM((CB,), jnp.int32),
            pltpu.VMEM((G,), jnp.int32),
            pltpu.VMEM((G,), jnp.int32),
            pltpu.VMEM((G,), jnp.int32),
            pltpu.VMEM((CB, d), jnp.float32),
            pltpu.VMEM((G, d), jnp.float32),
            pltpu.VMEM((G, d), jnp.float32),
            pltpu.VMEM((G, d), jnp.float32),
            pltpu.VMEM((CB, d), jnp.float32),
            pltpu.SemaphoreType.DMA,
        ],
    )
    def sc_gather(idb, ncell, ndrug, ngene, pgene, pdrug, ctab,
                  out_center, out_c, out_d, out_g,
                  idx_id, idx_c, idx_d, idx_g,
                  rows_ctr, rows_c, rows_d, rows_g, acc, sem):
        wid = lax.axis_index("s") * info.num_cores + lax.axis_index("c")
        base = wid * rows_w

        def chunk(ci, carry):
            off = pl.multiple_of(base + ci * CB, CB)
            noff = pl.multiple_of(off * s_per, G)
            pltpu.sync_copy(idb.at[pl.ds(off, CB)], idx_id)
            pltpu.sync_copy(ncell.at[pl.ds(noff, G)], idx_c)
            pltpu.sync_copy(ndrug.at[pl.ds(noff, G)], idx_d)
            pltpu.sync_copy(ngene.at[pl.ds(noff, G)], idx_g)
            cp0 = pltpu.async_copy(pgene.at[idx_id], rows_ctr, sem)
            cp1 = pltpu.async_copy(ctab.at[idx_c], rows_c, sem)
            cp2 = pltpu.async_copy(pdrug.at[idx_d], rows_d, sem)
            cp3 = pltpu.async_copy(pgene.at[idx_g], rows_g, sem)
            cp0.wait()
            cp1.wait()
            cp2.wait()
            cp3.wait()
            pltpu.sync_copy(rows_ctr, out_center.at[pl.ds(off, CB)])

            def reduce_into(rows, out_ref):
                def per_center(i, c2):
                    for g in range(d // 16):
                        sl = pl.ds(g * 16, 16)
                        v = rows[i * s_per, sl]
                        for j in range(1, s_per):
                            v = v + rows[i * s_per + j, sl]
                        acc[i, sl] = v
                    return c2
                lax.fori_loop(0, CB, per_center, 0)
                pltpu.sync_copy(acc, out_ref.at[pl.ds(off, CB)])

            reduce_into(rows_c, out_c)
            reduce_into(rows_d, out_d)
            reduce_into(rows_g, out_g)
            return carry

        lax.fori_loop(0, nchunks, chunk, 0)

    return sc_gather


def _combine_body(s_per, d, center_ref, sc_ref, sd_ref, sg_ref, att_ref, o_ref):
    inv = 1.0 / s_per
    center = center_ref[...]
    aggc = sc_ref[...] * inv
    aggd = sd_ref[...] * inv
    aggg = sg_ref[...] * inv
    att = att_ref[...]
    a1 = att[:, :d]
    a2 = att[:, d:]
    base = lax.dot_general(center, a1, (((1,), (1,)), ((), ())),
                           preferred_element_type=jnp.float32)
    q0 = jnp.sum(center * a2[0][None, :], axis=1, keepdims=True)
    q1 = jnp.sum(aggc * a2[1][None, :], axis=1, keepdims=True)
    q2 = jnp.sum(aggd * a2[2][None, :], axis=1, keepdims=True)
    q3 = jnp.sum(aggg * a2[3][None, :], axis=1, keepdims=True)
    s = base + jnp.concatenate([q0, q1, q2, q3], axis=1)
    s = jnp.where(s >= 0, s, 0.2 * s)
    m = jnp.max(s, axis=1, keepdims=True)
    e = jnp.exp(s - m)
    a = e / jnp.sum(e, axis=1, keepdims=True)
    o_ref[...] = (a[:, 0:1] * center + a[:, 1:2] * aggc
                  + a[:, 2:3] * aggd + a[:, 3:4] * aggg)


def _combine(center, sum_c, sum_d, sum_g, att, s_per):
    Bn, d = center.shape
    blk = 2048
    assert Bn % blk == 0
    body = functools.partial(_combine_body, s_per, d)
    return pl.pallas_call(
        body,
        grid=(Bn // blk,),
        in_specs=[pl.BlockSpec((blk, d), lambda i: (i, 0))] * 4
        + [pl.BlockSpec(att.shape, lambda i: (0, 0))],
        out_specs=pl.BlockSpec((blk, d), lambda i: (i, 0)),
        out_shape=jax.ShapeDtypeStruct((Bn, d), jnp.float32),
    )(center, sum_c, sum_d, sum_g, att)


def kernel(id_batch, neigh_cell, neigh_drug, neigh_gene, drug_features,
           gene_features, cell_table, W_drug, b_drug, W_gene, b_gene, att):
    Bn, s_per = neigh_cell.shape
    d = W_gene.shape[1]
    pgene = _project(gene_features, W_gene, b_gene)
    pdrug = _project(drug_features, W_drug, b_drug)
    sc = _make_sc_gather(Bn, d, s_per)
    center, sum_c, sum_d, sum_g = sc(
        id_batch.astype(jnp.int32),
        neigh_cell.reshape(-1).astype(jnp.int32),
        neigh_drug.reshape(-1).astype(jnp.int32),
        neigh_gene.reshape(-1).astype(jnp.int32),
        pgene, pdrug, cell_table)
    return _combine(center, sum_c, sum_d, sum_g, att, s_per)


# speedup vs baseline: 7.4063x; 1.6155x over previous
import functools

import jax
import jax.numpy as jnp
from jax import lax
from jax.experimental import pallas as pl
from jax.experimental.pallas import tpu as pltpu
from jax.experimental.pallas import tpu_sc as plsc


def _proj_body(x_ref, w_ref, b_ref, o_ref):
    o_ref[...] = (
        jnp.dot(x_ref[...], w_ref[...], preferred_element_type=jnp.float32)
        + b_ref[...]
    )


def _project(x, w, b):
    n, fdim = x.shape
    d = w.shape[1]
    blk = 2000
    assert n % blk == 0
    return pl.pallas_call(
        _proj_body,
        grid=(n // blk,),
        in_specs=[
            pl.BlockSpec((blk, fdim), lambda i: (i, 0)),
            pl.BlockSpec((fdim, d), lambda i: (0, 0)),
            pl.BlockSpec((1, d), lambda i: (0, 0)),
        ],
        out_specs=pl.BlockSpec((blk, d), lambda i: (i, 0)),
        out_shape=jax.ShapeDtypeStruct((n, d), jnp.float32),
    )(x, w, b.reshape(1, d))


def _make_sc_gather(Bn, d, s_per):
    info = plsc.get_sparse_core_info()
    NW = info.num_cores * info.num_subcores
    CB = 8
    G = CB * s_per
    rows_w = Bn // NW
    nchunks = rows_w // CB
    assert Bn % (NW * CB) == 0 and nchunks % 2 == 0
    mesh = plsc.VectorSubcoreMesh(core_axis_name="c", subcore_axis_name="s")

    @functools.partial(
        pl.kernel,
        mesh=mesh,
        out_type=[jax.ShapeDtypeStruct((Bn, d), jnp.float32)] * 4,
        scratch_types=[
            pltpu.VMEM((rows_w,), jnp.int32),
            pltpu.VMEM((rows_w * s_per,), jnp.int32),
            pltpu.VMEM((rows_w * s_per,), jnp.int32),
            pltpu.VMEM((rows_w * s_per,), jnp.int32),
        ]
        + [pltpu.VMEM((CB, d), jnp.float32),
           pltpu.VMEM((G, d), jnp.float32),
           pltpu.VMEM((G, d), jnp.float32),
           pltpu.VMEM((G, d), jnp.float32)] * 2
        + [pltpu.VMEM((CB, d), jnp.float32)] * 6
        + [pltpu.SemaphoreType.DMA] * 4,
    )
    def sc_gather(idb, ncell, ndrug, ngene, pgene, pdrug, ctab,
                  out_center, out_c, out_d, out_g,
                  ixid, ixc, ixd, ixg,
                  rctr0, rc0, rd0, rg0, rctr1, rc1, rd1, rg1,
                  ac0, ad0, ag0, ac1, ad1, ag1,
                  sg0, sg1, so0, so1):
        wid = lax.axis_index("s") * info.num_cores + lax.axis_index("c")
        base = wid * rows_w
        rows = ((rctr0, rc0, rd0, rg0), (rctr1, rc1, rd1, rg1))
        accs = ((ac0, ad0, ag0), (ac1, ad1, ag1))
        sem_g = (sg0, sg1)
        sem_o = (so0, so1)

        pltpu.sync_copy(idb.at[pl.ds(base, rows_w)], ixid)
        nb = base * s_per
        pltpu.sync_copy(ncell.at[pl.ds(nb, rows_w * s_per)], ixc)
        pltpu.sync_copy(ndrug.at[pl.ds(nb, rows_w * s_per)], ixd)
        pltpu.sync_copy(ngene.at[pl.ds(nb, rows_w * s_per)], ixg)

        def fire_gathers(c, s):
            o1 = pl.multiple_of(c * CB, CB)
            o10 = pl.multiple_of(c * G, 8)
            pltpu.async_copy(pgene.at[ixid.at[pl.ds(o1, CB)]], rows[s][0],
                             sem_g[s])
            pltpu.async_copy(ctab.at[ixc.at[pl.ds(o10, G)]], rows[s][1],
                             sem_g[s])
            pltpu.async_copy(pdrug.at[ixd.at[pl.ds(o10, G)]], rows[s][2],
                             sem_g[s])
            pltpu.async_copy(pgene.at[ixg.at[pl.ds(o10, G)]], rows[s][3],
                             sem_g[s])

        def wait_gathers(s):
            pltpu.make_async_copy(pgene.at[pl.ds(0, CB)], rows[s][0],
                                  sem_g[s]).wait()
            pltpu.make_async_copy(ctab.at[pl.ds(0, G)], rows[s][1],
                                  sem_g[s]).wait()
            pltpu.make_async_copy(pdrug.at[pl.ds(0, G)], rows[s][2],
                                  sem_g[s]).wait()
            pltpu.make_async_copy(pgene.at[pl.ds(0, G)], rows[s][3],
                                  sem_g[s]).wait()

        def fire_out(c, s):
            off = pl.multiple_of(base + c * CB, CB)
            pltpu.async_copy(rows[s][0], out_center.at[pl.ds(off, CB)],
                             sem_o[s])
            pltpu.async_copy(accs[s][0], out_c.at[pl.ds(off, CB)], sem_o[s])
            pltpu.async_copy(accs[s][1], out_d.at[pl.ds(off, CB)], sem_o[s])
            pltpu.async_copy(accs[s][2], out_g.at[pl.ds(off, CB)], sem_o[s])

        def drain_out(s):
            pltpu.make_async_copy(out_center.at[pl.ds(0, CB)], rows[s][0],
                                  sem_o[s]).wait()
            pltpu.make_async_copy(out_c.at[pl.ds(0, CB)], accs[s][0],
                                  sem_o[s]).wait()
            pltpu.make_async_copy(out_d.at[pl.ds(0, CB)], accs[s][1],
                                  sem_o[s]).wait()
            pltpu.make_async_copy(out_g.at[pl.ds(0, CB)], accs[s][2],
                                  sem_o[s]).wait()

        def reduce(rows_t, acc_t):
            def per_center(i, carry):
                rbase = i * s_per
                for g in range(d // 16):
                    sl = pl.ds(g * 16, 16)
                    v = rows_t[rbase, sl]
                    for j in range(1, s_per):
                        v = v + rows_t[rbase + j, sl]
                    acc_t[i, sl] = v
                return carry
            lax.fori_loop(0, CB, per_center, 0)

        fire_gathers(0, 0)

        def step(cc, carry):
            for s in (0, 1):
                c = cc * 2 + s
                if s == 0:
                    @pl.when(cc > 0)
                    def _():
                        drain_out(1)
                    fire_gathers(c + 1, 1)
                else:
                    drain_out(0)

                    @pl.when(cc < nchunks // 2 - 1)
                    def _():
                        fire_gathers(c + 1, 0)
                wait_gathers(s)
                reduce(rows[s][1], accs[s][0])
                reduce(rows[s][2], accs[s][1])
                reduce(rows[s][3], accs[s][2])
                fire_out(c, s)
            return carry

        lax.fori_loop(0, nchunks // 2, step, 0)
        drain_out(1)

    return sc_gather


def _combine_body(s_per, d, center_ref, sc_ref, sd_ref, sg_ref, att_ref, o_ref):
    inv = 1.0 / s_per
    center = center_ref[...]
    aggc = sc_ref[...] * inv
    aggd = sd_ref[...] * inv
    aggg = sg_ref[...] * inv
    att = att_ref[...]
    a1 = att[:, :d]
    a2 = att[:, d:]
    base = lax.dot_general(center, a1, (((1,), (1,)), ((), ())),
                           preferred_element_type=jnp.float32)
    q0 = jnp.sum(center * a2[0][None, :], axis=1, keepdims=True)
    q1 = jnp.sum(aggc * a2[1][None, :], axis=1, keepdims=True)
    q2 = jnp.sum(aggd * a2[2][None, :], axis=1, keepdims=True)
    q3 = jnp.sum(aggg * a2[3][None, :], axis=1, keepdims=True)
    s = base + jnp.concatenate([q0, q1, q2, q3], axis=1)
    s = jnp.where(s >= 0, s, 0.2 * s)
    m = jnp.max(s, axis=1, keepdims=True)
    e = jnp.exp(s - m)
    a = e / jnp.sum(e, axis=1, keepdims=True)
    o_ref[...] = (a[:, 0:1] * center + a[:, 1:2] * aggc
                  + a[:, 2:3] * aggd + a[:, 3:4] * aggg)


def _combine(center, sum_c, sum_d, sum_g, att, s_per):
    Bn, d = center.shape
    blk = 2048
    assert Bn % blk == 0
    body = functools.partial(_combine_body, s_per, d)
    return pl.pallas_call(
        body,
        grid=(Bn // blk,),
        in_specs=[pl.BlockSpec((blk, d), lambda i: (i, 0))] * 4
        + [pl.BlockSpec(att.shape, lambda i: (0, 0))],
        out_specs=pl.BlockSpec((blk, d), lambda i: (i, 0)),
        out_shape=jax.ShapeDtypeStruct((Bn, d), jnp.float32),
    )(center, sum_c, sum_d, sum_g, att)


def kernel(id_batch, neigh_cell, neigh_drug, neigh_gene, drug_features,
           gene_features, cell_table, W_drug, b_drug, W_gene, b_gene, att):
    Bn, s_per = neigh_cell.shape
    d = W_gene.shape[1]
    pgene = _project(gene_features, W_gene, b_gene)
    pdrug = _project(drug_features, W_drug, b_drug)
    sc = _make_sc_gather(Bn, d, s_per)
    center, sum_c, sum_d, sum_g = sc(
        id_batch.astype(jnp.int32),
        neigh_cell.reshape(-1).astype(jnp.int32),
        neigh_drug.reshape(-1).astype(jnp.int32),
        neigh_gene.reshape(-1).astype(jnp.int32),
        pgene, pdrug, cell_table)
    return _combine(center, sum_c, sum_d, sum_g, att, s_per)


# speedup vs baseline: 7.5137x; 1.0145x over previous
import functools

import jax
import jax.numpy as jnp
from jax import lax
from jax.experimental import pallas as pl
from jax.experimental.pallas import tpu as pltpu
from jax.experimental.pallas import tpu_sc as plsc


def _proj_body(x_ref, w_ref, b_ref, o_ref):
    o_ref[...] = (
        jnp.dot(x_ref[...], w_ref[...], preferred_element_type=jnp.float32)
        + b_ref[...]
    )


def _project(x, w, b):
    n, fdim = x.shape
    d = w.shape[1]
    blk = 2000
    assert n % blk == 0
    return pl.pallas_call(
        _proj_body,
        grid=(n // blk,),
        in_specs=[
            pl.BlockSpec((blk, fdim), lambda i: (i, 0)),
            pl.BlockSpec((fdim, d), lambda i: (0, 0)),
            pl.BlockSpec((1, d), lambda i: (0, 0)),
        ],
        out_specs=pl.BlockSpec((blk, d), lambda i: (i, 0)),
        out_shape=jax.ShapeDtypeStruct((n, d), jnp.float32),
    )(x, w, b.reshape(1, d))


def _make_sc_gather(Bn, d, s_per, streams):
    info = plsc.get_sparse_core_info()
    NW = info.num_cores * info.num_subcores
    CB = 8
    G = CB * s_per
    rows_w = Bn // NW
    nchunks = rows_w // CB
    assert Bn % (NW * CB) == 0 and nchunks % 2 == 0
    mesh = plsc.VectorSubcoreMesh(core_axis_name="c", subcore_axis_name="s")
    ns = len(streams)

    idx_scratch = [
        pltpu.VMEM((rows_w * (s_per if st["reduce"] else 1),), jnp.int32)
        for st in streams
    ]
    row_scratch = [
        pltpu.VMEM(((G if st["reduce"] else CB), d), jnp.float32)
        for st in streams
    ] * 2
    acc_scratch = [pltpu.VMEM((CB, d), jnp.float32)
                   for st in streams if st["reduce"]] * 2
    nred = sum(1 for st in streams if st["reduce"])

    @functools.partial(
        pl.kernel,
        mesh=mesh,
        out_type=[jax.ShapeDtypeStruct((Bn, d), jnp.float32)] * ns,
        scratch_types=idx_scratch + row_scratch + acc_scratch
        + [pltpu.SemaphoreType.DMA] * 4,
    )
    def sc_gather(*refs):
        tabs = refs[:ns]
        idxs = refs[ns:2 * ns]
        outs = refs[2 * ns:3 * ns]
        k = 3 * ns
        ix = refs[k:k + ns]
        rows = (refs[k + ns:k + 2 * ns], refs[k + 2 * ns:k + 3 * ns])
        k2 = k + 3 * ns
        accs = (refs[k2:k2 + nred], refs[k2 + nred:k2 + 2 * nred])
        sem_g = refs[k2 + 2 * nred:k2 + 2 * nred + 2]
        sem_o = refs[k2 + 2 * nred + 2:k2 + 2 * nred + 4]

        wid = lax.axis_index("s") * info.num_cores + lax.axis_index("c")
        base = wid * rows_w

        for t, st in enumerate(streams):
            rep = s_per if st["reduce"] else 1
            pltpu.sync_copy(idxs[t].at[pl.ds(base * rep, rows_w * rep)],
                            ix[t])

        def fire_gathers(c, s):
            for t, st in enumerate(streams):
                if st["reduce"]:
                    o = pl.multiple_of(c * G, 8)
                    n = G
                else:
                    o = pl.multiple_of(c * CB, CB)
                    n = CB
                pltpu.async_copy(tabs[t].at[ix[t].at[pl.ds(o, n)]],
                                 rows[s][t], sem_g[s])

        def wait_gathers(s):
            for t, st in enumerate(streams):
                n = G if st["reduce"] else CB
                pltpu.make_async_copy(tabs[t].at[pl.ds(0, n)], rows[s][t],
                                      sem_g[s]).wait()

        def fire_out(c, s):
            off = pl.multiple_of(base + c * CB, CB)
            r = 0
            for t, st in enumerate(streams):
                src = accs[s][r] if st["reduce"] else rows[s][t]
                if st["reduce"]:
                    r += 1
                pltpu.async_copy(src, outs[t].at[pl.ds(off, CB)], sem_o[s])

        def drain_out(s):
            r = 0
            for t, st in enumerate(streams):
                dst = accs[s][r] if st["reduce"] else rows[s][t]
                if st["reduce"]:
                    r += 1
                pltpu.make_async_copy(outs[t].at[pl.ds(0, CB)], dst,
                                      sem_o[s]).wait()

        def reduce(rows_t, acc_t):
            def per_center(i, carry):
                rbase = i * s_per
                for g in range(d // 16):
                    sl = pl.ds(g * 16, 16)
                    v = rows_t[rbase, sl]
                    for j in range(1, s_per):
                        v = v + rows_t[rbase + j, sl]
                    acc_t[i, sl] = v
                return carry
            lax.fori_loop(0, CB, per_center, 0)

        def reduce_all(s):
            r = 0
            for t, st in enumerate(streams):
                if st["reduce"]:
                    reduce(rows[s][t], accs[s][r])
                    r += 1

        fire_gathers(0, 0)

        def step(cc, carry):
            for s in (0, 1):
                c = cc * 2 + s
                if s == 0:
                    @pl.when(cc > 0)
                    def _():
                        drain_out(1)
                    fire_gathers(c + 1, 1)
                else:
                    drain_out(0)

                    @pl.when(cc < nchunks // 2 - 1)
                    def _():
                        fire_gathers(c + 1, 0)
                wait_gathers(s)
                reduce_all(s)
                fire_out(c, s)
            return carry

        lax.fori_loop(0, nchunks // 2, step, 0)
        drain_out(1)

    return sc_gather


def _combine_body(s_per, d, center_ref, sc_ref, sd_ref, sg_ref, att_ref, o_ref):
    inv = 1.0 / s_per
    center = center_ref[...]
    aggc = sc_ref[...] * inv
    aggd = sd_ref[...] * inv
    aggg = sg_ref[...] * inv
    att = att_ref[...]
    a1 = att[:, :d]
    a2 = att[:, d:]
    base = lax.dot_general(center, a1, (((1,), (1,)), ((), ())),
                           preferred_element_type=jnp.float32)
    q0 = jnp.sum(center * a2[0][None, :], axis=1, keepdims=True)
    q1 = jnp.sum(aggc * a2[1][None, :], axis=1, keepdims=True)
    q2 = jnp.sum(aggd * a2[2][None, :], axis=1, keepdims=True)
    q3 = jnp.sum(aggg * a2[3][None, :], axis=1, keepdims=True)
    s = base + jnp.concatenate([q0, q1, q2, q3], axis=1)
    s = jnp.where(s >= 0, s, 0.2 * s)
    m = jnp.max(s, axis=1, keepdims=True)
    e = jnp.exp(s - m)
    a = e / jnp.sum(e, axis=1, keepdims=True)
    o_ref[...] = (a[:, 0:1] * center + a[:, 1:2] * aggc
                  + a[:, 2:3] * aggd + a[:, 3:4] * aggg)


def _combine(center, sum_c, sum_d, sum_g, att, s_per):
    Bn, d = center.shape
    blk = 2048
    assert Bn % blk == 0
    body = functools.partial(_combine_body, s_per, d)
    return pl.pallas_call(
        body,
        grid=(Bn // blk,),
        in_specs=[pl.BlockSpec((blk, d), lambda i: (i, 0))] * 4
        + [pl.BlockSpec(att.shape, lambda i: (0, 0))],
        out_specs=pl.BlockSpec((blk, d), lambda i: (i, 0)),
        out_shape=jax.ShapeDtypeStruct((Bn, d), jnp.float32),
    )(center, sum_c, sum_d, sum_g, att)


def kernel(id_batch, neigh_cell, neigh_drug, neigh_gene, drug_features,
           gene_features, cell_table, W_drug, b_drug, W_gene, b_gene, att):
    Bn, s_per = neigh_cell.shape
    d = W_gene.shape[1]
    sc_cell = _make_sc_gather(Bn, d, s_per, [{"reduce": True}])
    (sum_c,) = sc_cell(cell_table,
                       neigh_cell.reshape(-1).astype(jnp.int32))
    pgene = _project(gene_features, W_gene, b_gene)
    pdrug = _project(drug_features, W_drug, b_drug)
    sc_main = _make_sc_gather(
        Bn, d, s_per,
        [{"reduce": False}, {"reduce": True}, {"reduce": True}])
    center, sum_d, sum_g = sc_main(
        pgene, pdrug, pgene,
        id_batch.astype(jnp.int32),
        neigh_drug.reshape(-1).astype(jnp.int32),
        neigh_gene.reshape(-1).astype(jnp.int32))
    return _combine(center, sum_c, sum_d, sum_g, att, s_per)
